# Initial kernel scaffold; baseline (speedup 1.0000x reference)
import jax
import jax.numpy as jnp
from jax.experimental import pallas as pl


def kernel(x, edge_index, edge_attr, W, b):
    raise NotImplementedError("write your pallas kernel here")


# speedup vs baseline: 3.9674x; 3.9674x over previous
import functools

import jax
import jax.numpy as jnp
from jax import lax
from jax.experimental import pallas as pl
from jax.experimental.pallas import tpu as pltpu
from jax.experimental.pallas import tpu_sc as plsc

N = 10000
E = 320000
D = 128

NC = 2
NS = 16
NW = NC * NS
EPW = E // NW
C = 80
NCHUNK = EPW // C
NPAD = 10240
NPS = NPAD // NS
ZR = 128

_mesh = plsc.VectorSubcoreMesh(
    core_axis_name="c", subcore_axis_name="s", num_cores=NC, num_subcores=NS)


@functools.partial(
    pl.kernel,
    out_type=jax.ShapeDtypeStruct((NC, NPAD, D), jnp.float32),
    mesh=_mesh,
    scratch_types=[
        pltpu.VMEM((ZR, D), jnp.float32),
        pltpu.VMEM((C,), jnp.int32),
        pltpu.VMEM((C,), jnp.int32),
        pltpu.VMEM((C, D), jnp.float32),
        pltpu.VMEM((C, D), jnp.float32),
        pltpu.VMEM_SHARED((NPAD, D), jnp.float32),
        pltpu.SemaphoreType.DMA,
    ],
)
def _sc_aggregate(src_hbm, dst_hbm, ea_hbm, x_hbm, part_hbm,
                  zbuf, sidx, didx, xr, ea, agg, gsem):
    cid = lax.axis_index("c")
    sid = lax.axis_index("s")
    wid = sid * NC + cid

    zeros16 = jnp.zeros((16,), jnp.float32)

    def _zero_row(r, _):
        for j in range(D // 16):
            zbuf[r, pl.ds(j * 16, 16)] = zeros16
        return ()

    lax.fori_loop(0, ZR, _zero_row, (), unroll=False)
    for k in range(NPS // ZR):
        pltpu.sync_copy(zbuf, agg.at[pl.ds(sid * NPS + k * ZR, ZR), :])
    plsc.subcore_barrier()

    base = wid * EPW

    def _chunk(ci, _):
        off = base + ci * C
        pltpu.sync_copy(src_hbm.at[pl.ds(off, C)], sidx)
        pltpu.sync_copy(dst_hbm.at[pl.ds(off, C)], didx)
        gcp = pltpu.async_copy(x_hbm.at[sidx], xr, gsem)
        pltpu.sync_copy(ea_hbm.at[pl.ds(off, C), :], ea)
        gcp.wait()

        def _row(r, _):
            for j in range(D // 16):
                s = pl.ds(j * 16, 16)
                xr[r, s] = jnp.maximum(xr[r, s] + ea[r, s], 0.0)
            return ()

        lax.fori_loop(0, C, _row, (), unroll=False)
        pltpu.sync_copy(xr, agg.at[didx], add=True)
        return ()

    lax.fori_loop(0, NCHUNK, _chunk, (), unroll=False)

    plsc.subcore_barrier()
    pltpu.sync_copy(agg.at[pl.ds(sid * NPS, NPS), :],
                    part_hbm.at[cid, pl.ds(sid * NPS, NPS), :])


_BN = 1000


def _tc_linear_body(x_ref, p0_ref, p1_ref, w_ref, b_ref, o_ref):
    h = x_ref[...] + p0_ref[...] + p1_ref[...]
    o_ref[...] = lax.dot_general(
        h, w_ref[...], (((1,), (1,)), ((), ())),
        preferred_element_type=jnp.float32) + b_ref[...]


def _tc_linear(x, p0, p1, w, b2):
    return pl.pallas_call(
        _tc_linear_body,
        grid=(N // _BN,),
        in_specs=[
            pl.BlockSpec((_BN, D), lambda i: (i, 0)),
            pl.BlockSpec((_BN, D), lambda i: (i, 0)),
            pl.BlockSpec((_BN, D), lambda i: (i, 0)),
            pl.BlockSpec((D, D), lambda i: (0, 0)),
            pl.BlockSpec((1, D), lambda i: (0, 0)),
        ],
        out_specs=pl.BlockSpec((_BN, D), lambda i: (i, 0)),
        out_shape=jax.ShapeDtypeStruct((N, D), jnp.float32),
    )(x, p0, p1, w, b2)


def kernel(x, edge_index, edge_attr, W, b):
    src = edge_index[0]
    dst = edge_index[1]
    part = _sc_aggregate(src, dst, edge_attr, x)
    return _tc_linear(x, part[0, :N], part[1, :N], W, b.reshape(1, D))
